# Initial kernel scaffold; baseline (speedup 1.0000x reference)
import jax
import jax.numpy as jnp
from jax.experimental import pallas as pl


def kernel(pos, edge_index, W_l1, b_l1, W_r1, W_se1, W_l2, b_l2, W_r2, W_se2, W_m3, b_m3, W_m4, b_m4, alpha):
    raise NotImplementedError("write your pallas kernel here")


# speedup vs baseline: 12.0057x; 12.0057x over previous
import functools
import math

import jax
import jax.numpy as jnp
from jax import lax
from jax.experimental import pallas as pl
from jax.experimental.pallas import tpu as pltpu
from jax.experimental.pallas import tpu_sc as plsc

N = 100000
E = 1600000
H = 32
HH = H // 2
NPAD = 100352

NC = 2
NS = 16
NW = NC * NS

ROWS_PER_S = NPAD // NS

LW = 128
EPAD = 1605632
IDXROWS = EPAD // LW
ROWS_W = IDXROWS // NW
KCH = 8
NCH = ROWS_W // KCH


def _make_seg_body(F, stage_table):
    def body(src2, dst2, tab, zf, out0, out1, *refs):
        if stage_table:
            table_s, acc_s, src_v, dst_v, rows_v, sem = refs
        else:
            acc_s, src_v, dst_v, rows_v, sem = refs
        c = lax.axis_index("c")
        s = lax.axis_index("s")
        w = c * NS + s
        r0 = s * ROWS_PER_S
        if stage_table:
            pltpu.sync_copy(tab.at[pl.ds(r0, ROWS_PER_S), :],
                            table_s.at[pl.ds(r0, ROWS_PER_S), :])
            gsrc = table_s
        else:
            gsrc = tab
        pltpu.sync_copy(zf.at[pl.ds(r0, ROWS_PER_S), :],
                        acc_s.at[pl.ds(r0, ROWS_PER_S), :])
        plsc.subcore_barrier()
        row0 = w * ROWS_W

        def it(k, carry):
            rb = row0 + k * KCH
            pltpu.sync_copy(src2.at[pl.ds(rb, KCH), :], src_v)
            pltpu.sync_copy(dst2.at[pl.ds(rb, KCH), :], dst_v)
            gets = [pltpu.async_copy(gsrc.at[src_v.at[j]], rows_v.at[j], sem)
                    for j in range(KCH)]
            for j in range(KCH):
                gets[j].wait()
            for j in range(KCH):
                pltpu.sync_copy(rows_v.at[j], acc_s.at[dst_v.at[j]], add=True)
            return carry

        lax.fori_loop(0, NCH, it, 0)
        plsc.subcore_barrier()

        @pl.when(c == 0)
        def _():
            pltpu.sync_copy(acc_s.at[pl.ds(r0, ROWS_PER_S), :],
                            out0.at[pl.ds(r0, ROWS_PER_S), :])

        @pl.when(c == 1)
        def _():
            pltpu.sync_copy(acc_s.at[pl.ds(r0, ROWS_PER_S), :],
                            out1.at[pl.ds(r0, ROWS_PER_S), :])

    return body


def _make_seg(F, stage_table):
    scratch = []
    if stage_table:
        scratch.append(pltpu.VMEM_SHARED((NPAD, F), jnp.float32))
    scratch += [
        pltpu.VMEM_SHARED((NPAD, F), jnp.float32),
        pltpu.VMEM((KCH, LW), jnp.int32),
        pltpu.VMEM((KCH, LW), jnp.int32),
        pltpu.VMEM((KCH, LW, F), jnp.float32),
        pltpu.SemaphoreType.DMA,
    ]
    return pl.kernel(
        _make_seg_body(F, stage_table),
        out_type=[jax.ShapeDtypeStruct((NPAD, F), jnp.float32)] * 2,
        mesh=plsc.VectorSubcoreMesh(core_axis_name="c", subcore_axis_name="s",
                                    num_cores=NC, num_subcores=NS),
        scratch_types=scratch,
        compiler_params=pltpu.CompilerParams(use_tc_tiling_on_sc=False),
    )


F1 = 8


@functools.cache
def _get_seg1():
    return _make_seg(F1, False)


@functools.cache
def _get_seg2():
    return _make_seg(HH, False)


R = 2048
G = NPAD // R
ISQH = 1.0 / math.sqrt(float(H))


def _tc1_body(p0, p1, x2, pk, wse, h1a, h1b):
    agg = p0[...] + p1[...]
    x = x2[...]
    agg = agg[:, :2]
    x = x[:, :2]
    z = (agg[:, 0:1] * pk[0:1, :] + agg[:, 1:2] * pk[1:2, :]
         + x[:, 0:1] * pk[2:3, :] + x[:, 1:2] * pk[3:4, :] + pk[4:5, :])
    z = jnp.where(z >= 0, z, 0.01 * z)
    h1 = jnp.dot(z, wse[...], preferred_element_type=jnp.float32) * ISQH
    h1a[...] = h1[:, :HH]
    h1b[...] = h1[:, HH:]


_tc1 = pl.pallas_call(
    _tc1_body,
    grid=(G,),
    in_specs=[
        pl.BlockSpec((R, F1), lambda i: (i, 0)),
        pl.BlockSpec((R, F1), lambda i: (i, 0)),
        pl.BlockSpec((R, F1), lambda i: (i, 0)),
        pl.BlockSpec((8, H), lambda i: (0, 0)),
        pl.BlockSpec((H, H), lambda i: (0, 0)),
    ],
    out_specs=[pl.BlockSpec((R, HH), lambda i: (i, 0)),
               pl.BlockSpec((R, HH), lambda i: (i, 0))],
    out_shape=[jax.ShapeDtypeStruct((NPAD, HH), jnp.float32)] * 2,
)


def _tc2_body(a2a0, a2a1, a2b0, a2b1, h1a, h1b, wl, wr, wse, wm3, pk, out):
    h1 = jnp.concatenate([h1a[...], h1b[...]], axis=1)
    agg = jnp.concatenate([a2a0[...] + a2a1[...], a2b0[...] + a2b1[...]],
                          axis=1)
    z = (jnp.dot(agg, wl[...], preferred_element_type=jnp.float32)
         + jnp.dot(h1, wr[...], preferred_element_type=jnp.float32)
         + pk[0:1, :])
    z = jnp.where(z >= 0, z, 0.01 * z)
    h2 = jnp.dot(z, wse[...], preferred_element_type=jnp.float32) * ISQH
    skip = pk[3, 1] * h1 + h2
    o = jnp.maximum(
        jnp.dot(skip, wm3[...], preferred_element_type=jnp.float32)
        + pk[1:2, :], 0.0)
    out[...] = jnp.sum(o * pk[2:3, :], axis=1, keepdims=True) + pk[3, 0]


_tc2 = pl.pallas_call(
    _tc2_body,
    grid=(G,),
    in_specs=[
        pl.BlockSpec((R, HH), lambda i: (i, 0)),
        pl.BlockSpec((R, HH), lambda i: (i, 0)),
        pl.BlockSpec((R, HH), lambda i: (i, 0)),
        pl.BlockSpec((R, HH), lambda i: (i, 0)),
        pl.BlockSpec((R, HH), lambda i: (i, 0)),
        pl.BlockSpec((R, HH), lambda i: (i, 0)),
        pl.BlockSpec((H, H), lambda i: (0, 0)),
        pl.BlockSpec((H, H), lambda i: (0, 0)),
        pl.BlockSpec((H, H), lambda i: (0, 0)),
        pl.BlockSpec((H, H), lambda i: (0, 0)),
        pl.BlockSpec((8, H), lambda i: (0, 0)),
    ],
    out_specs=pl.BlockSpec((R, 1), lambda i: (i, 0)),
    out_shape=jax.ShapeDtypeStruct((NPAD, 1), jnp.float32),
)


def kernel(pos, edge_index, W_l1, b_l1, W_r1, W_se1, W_l2, b_l2, W_r2, W_se2,
           W_m3, b_m3, W_m4, b_m4, alpha):
    x2 = jnp.pad(pos[:, :2], ((0, NPAD - N), (0, F1 - 2)))
    z2 = jnp.zeros((NPAD, F1), jnp.float32)
    z16 = jnp.zeros((NPAD, HH), jnp.float32)
    pad = EPAD - E
    padidx = N + (jnp.arange(pad, dtype=jnp.int32) % LW)
    src2 = jnp.concatenate([edge_index[0], padidx]).reshape(IDXROWS, LW)
    dst2 = jnp.concatenate([edge_index[1], padidx]).reshape(IDXROWS, LW)

    p0, p1 = _get_seg1()(src2, dst2, x2, z2)

    pk1 = jnp.concatenate(
        [W_l1, W_r1, b_l1[None, :], jnp.zeros((3, H), jnp.float32)], axis=0)
    h1a, h1b = _tc1(p0, p1, x2, pk1, W_se1)

    seg2 = _get_seg2()
    a2a0, a2a1 = seg2(src2, dst2, h1a, z16)
    a2b0, a2b1 = seg2(src2, dst2, h1b, z16)

    misc = jnp.zeros((H,), jnp.float32).at[0].set(b_m4[0]).at[1].set(alpha)
    pk2 = jnp.concatenate(
        [b_l2[None, :], b_m3[None, :], W_m4.T, misc[None, :],
         jnp.zeros((4, H), jnp.float32)], axis=0)
    pred = _tc2(a2a0, a2a1, a2b0, a2b1, h1a, h1b, W_l2, W_r2, W_se2, W_m3,
                pk2)
    return pred[:N, 0]
